# baseline (speedup 1.0000x reference)
import jax
import jax.numpy as jnp
from jax import lax
from jax.experimental import pallas as pl
from jax.experimental.pallas import tpu as pltpu
from jax.experimental.pallas import tpu_sc as plsc

_TOKENS = 8192
_D = 32
_E = 4
_H = _D // 16
_NW = 32
_TPW = _TOKENS // _NW
_NG = _TPW // 16
_PAIRS = [(0, 1), (0, 2), (0, 3), (1, 2), (1, 3), (2, 3)]
_RS = _D + 1
_PS = _D * _D + 1

_i32 = jnp.int32
_f32 = jnp.float32


def _splat(v, dtype=_i32):
    return jnp.full((16,), v, dtype)


def _tree_sum(vals):
    while len(vals) > 1:
        vals = [vals[i] + vals[i + 1] for i in range(0, len(vals), 2)]
    return vals[0]


def _round_bf16(v):
    i = plsc.bitcast(v, _i32)
    r = i + 0x7FFF + (lax.shift_right_logical(i, 16) & 1)
    return plsc.bitcast(r & jnp.int32(-65536), _f32)


def _sc_body(x_hbm, gw_hbm, ew_hbm, out_hbm,
             x_raw, x_v, gate_v, ew_v, wp_v, out_v, out_raw):
    wid = lax.axis_index("s") * 2 + lax.axis_index("c")
    base = wid * _TPW * _D
    pltpu.sync_copy(x_hbm.at[pl.ds(base, _TPW * _D)], x_raw)
    pltpu.sync_copy(gw_hbm, gate_v)
    pltpu.sync_copy(ew_hbm, ew_v)

    def pad_row(t, carry):
        for h in range(_H):
            x_v[pl.ds(t * _RS + h * 16, 16)] = \
                x_raw[pl.ds(t * _D + h * 16, 16)]
        return carry
    lax.fori_loop(0, _TPW, pad_row, 0)

    for p, (a, b) in enumerate(_PAIRS):
        for o in range(_D):
            for h in range(_H):
                off = o * _D + h * 16
                wa = ew_v[pl.ds(a * 1024 + off, 16)]
                wb = ew_v[pl.ds(b * 1024 + off, 16)]
                wp_v[pl.ds(p * _PS + off, 16)] = wa + wb

    lane16 = lax.iota(_i32, 16)

    gs = [[None] * _D for _ in range(_E)]
    for e in range(_E):
        for h in range(_H):
            row = _round_bf16(gate_v[pl.ds(e * _D + h * 16, 16)])
            for k in range(16):
                gs[e][h * 16 + k] = jnp.sum(
                    jnp.where(lane16 == k, row, jnp.float32(0.0)))

    def one_group(j):
        tokr = (j * 16 + lane16) * _RS
        xs = [plsc.load_gather(x_v, [tokr + _splat(d)]) for d in range(_D)]
        xbs = [_round_bf16(xs[d]) for d in range(_D)]
        ls = [_tree_sum([xbs[d] * gs[e][d] for d in range(_D)])
              for e in range(_E)]
        ms = []
        for e in range(_E):
            beat = _splat(0)
            for f in range(_E):
                if f == e:
                    continue
                c = (ls[f] >= ls[e]) if f < e else (ls[f] > ls[e])
                beat = beat + c.astype(_i32)
            ms.append((beat < 2).astype(_i32))
        n0 = 1 - ms[0]
        n1 = 1 - ms[1]
        lo = n0 + n0 * n1
        n3 = 1 - ms[3]
        n2 = 1 - ms[2]
        hi = 3 - n3 - n3 * n2
        pidx = lax.shift_right_logical(lo * (7 - lo), 1) + hi - lo - 1
        pbase = pidx * _PS
        for o in range(_D):
            acc = _tree_sum([
                xs[d] * plsc.load_gather(wp_v, [pbase + _splat(o * _D + d)])
                for d in range(_D)])
            plsc.store_scatter(out_v, [tokr + _splat(o)], acc)

    def group2(j, carry):
        one_group(j * 2)
        one_group(j * 2 + 1)
        return carry

    lax.fori_loop(0, _NG // 2, group2, 0)

    def unpad_row(t, carry):
        for h in range(_H):
            out_raw[pl.ds(t * _D + h * 16, 16)] = \
                out_v[pl.ds(t * _RS + h * 16, 16)]
        return carry
    lax.fori_loop(0, _TPW, unpad_row, 0)
    pltpu.sync_copy(out_raw, out_hbm.at[pl.ds(base, _TPW * _D)])


@jax.jit
def kernel(x, gate_w, expert_w):
    mesh = plsc.VectorSubcoreMesh(core_axis_name="c", subcore_axis_name="s")
    run = pl.kernel(
        _sc_body,
        mesh=mesh,
        compiler_params=pltpu.CompilerParams(needs_layout_passes=False),
        out_type=jax.ShapeDtypeStruct((_TOKENS * _D,), _f32),
        scratch_types=[
            pltpu.VMEM((_TPW * _D,), _f32),
            pltpu.VMEM((_TPW * _RS,), _f32),
            pltpu.VMEM((_E * _D,), _f32),
            pltpu.VMEM((_E * _D * _D,), _f32),
            pltpu.VMEM((len(_PAIRS) * _PS,), _f32),
            pltpu.VMEM((_TPW * _RS,), _f32),
            pltpu.VMEM((_TPW * _D,), _f32),
        ],
    )
    out = run(x.reshape(_TOKENS * _D), gate_w.reshape(_E * _D),
              expert_w.reshape(_E * _D * _D))
    return out.reshape(_TOKENS, _D)


# speedup vs baseline: 1.1104x; 1.1104x over previous
import jax
import jax.numpy as jnp
from jax import lax
from jax.experimental import pallas as pl
from jax.experimental.pallas import tpu as pltpu
from jax.experimental.pallas import tpu_sc as plsc

_TOKENS = 8192
_D = 32
_E = 4
_H = _D // 16
_NW = 32
_TPW = _TOKENS // _NW
_NG = _TPW // 16
_PAIRS = [(0, 1), (0, 2), (0, 3), (1, 2), (1, 3), (2, 3)]
_RS = _D + 1
_PS = _D * _D + 1

_i32 = jnp.int32
_f32 = jnp.float32


def _splat(v, dtype=_i32):
    return jnp.full((16,), v, dtype)


def _tree_sum(vals):
    while len(vals) > 1:
        vals = [vals[i] + vals[i + 1] for i in range(0, len(vals), 2)]
    return vals[0]


def _round_bf16(v):
    i = plsc.bitcast(v, _i32)
    r = i + 0x7FFF + (lax.shift_right_logical(i, 16) & 1)
    return plsc.bitcast(r & jnp.int32(-65536), _f32)


def _sc_body(x_hbm, gw_hbm, ew_hbm, out_hbm,
             x_raw, x_v, gate_v, ew_v, wp_v, out_v, out_raw):
    wid = lax.axis_index("s") * 2 + lax.axis_index("c")
    base = wid * _TPW * _D
    pltpu.sync_copy(x_hbm.at[pl.ds(base, _TPW * _D)], x_raw)
    pltpu.sync_copy(gw_hbm, gate_v)
    pltpu.sync_copy(ew_hbm, ew_v)

    def pad_row(t, carry):
        for h in range(_H):
            x_v[pl.ds(t * _RS + h * 16, 16)] = \
                x_raw[pl.ds(t * _D + h * 16, 16)]
        return carry
    lax.fori_loop(0, _TPW, pad_row, 0)

    for p, (a, b) in enumerate(_PAIRS):
        for o in range(_D):
            for h in range(_H):
                off = o * _D + h * 16
                wa = ew_v[pl.ds(a * 1024 + off, 16)]
                wb = ew_v[pl.ds(b * 1024 + off, 16)]
                wp_v[pl.ds(p * _PS + off, 16)] = wa + wb

    lane16 = lax.iota(_i32, 16)

    gs = [[None] * _D for _ in range(_E)]
    for e in range(_E):
        for h in range(_H):
            row = _round_bf16(gate_v[pl.ds(e * _D + h * 16, 16)])
            for k in range(16):
                gs[e][h * 16 + k] = jnp.sum(
                    jnp.where(lane16 == k, row, jnp.float32(0.0)))

    def group(j, carry):
        tokr = (j * 16 + lane16) * _RS
        xs = [plsc.load_gather(x_v, [tokr + _splat(d)]) for d in range(_D)]
        xbs = [_round_bf16(xs[d]) for d in range(_D)]
        ls = [_tree_sum([xbs[d] * gs[e][d] for d in range(_D)])
              for e in range(_E)]
        ms = []
        for e in range(_E):
            beat = _splat(0)
            for f in range(_E):
                if f == e:
                    continue
                c = (ls[f] >= ls[e]) if f < e else (ls[f] > ls[e])
                beat = beat + c.astype(_i32)
            ms.append((beat < 2).astype(_i32))
        n0 = 1 - ms[0]
        n1 = 1 - ms[1]
        lo = n0 + n0 * n1
        n3 = 1 - ms[3]
        n2 = 1 - ms[2]
        hi = 3 - n3 - n3 * n2
        pidx = lax.shift_right_logical(lo * (7 - lo), 1) + hi - lo - 1
        pbase = pidx * _PS
        for o in range(_D):
            acc = _tree_sum([
                xs[d] * plsc.load_gather(wp_v, [pbase + _splat(o * _D + d)])
                for d in range(_D)])
            plsc.store_scatter(out_v, [tokr + _splat(o)], acc)
        return carry

    lax.fori_loop(0, _NG, group, 0)

    def unpad_row(t, carry):
        for h in range(_H):
            out_raw[pl.ds(t * _D + h * 16, 16)] = \
                out_v[pl.ds(t * _RS + h * 16, 16)]
        return carry
    lax.fori_loop(0, _TPW, unpad_row, 0)
    pltpu.sync_copy(out_raw, out_hbm.at[pl.ds(base, _TPW * _D)])


@jax.jit
def kernel(x, gate_w, expert_w):
    mesh = plsc.VectorSubcoreMesh(core_axis_name="c", subcore_axis_name="s")
    run = pl.kernel(
        _sc_body,
        mesh=mesh,
        compiler_params=pltpu.CompilerParams(needs_layout_passes=False),
        out_type=jax.ShapeDtypeStruct((_TOKENS * _D,), _f32),
        scratch_types=[
            pltpu.VMEM((_TPW * _D,), _f32),
            pltpu.VMEM((_TPW * _RS,), _f32),
            pltpu.VMEM((_E * _D,), _f32),
            pltpu.VMEM((_E * _D * _D,), _f32),
            pltpu.VMEM((len(_PAIRS) * _PS,), _f32),
            pltpu.VMEM((_TPW * _RS,), _f32),
            pltpu.VMEM((_TPW * _D,), _f32),
        ],
    )
    out = run(x.reshape(_TOKENS * _D), gate_w.reshape(_E * _D),
              expert_w.reshape(_E * _D * _D))
    return out.reshape(_TOKENS, _D)
